# baseline (speedup 1.0000x reference)
import functools

import jax
import jax.numpy as jnp
from jax import lax
from jax.experimental import pallas as pl
from jax.experimental.pallas import tpu as pltpu
import jax.experimental.pallas.tpu_sc as plsc

SELF_C = 1.0

NC = 2
NS = 16
NW = NC * NS

CHUNK = 128


def _zero_vmem(buf, rows, width, dtype=jnp.float32):
  z = jnp.zeros((16,), dtype)
  for i in range(rows):
    for j in range(width // 16):
      buf[i, pl.ds(j * 16, 16)] = z


def _make_deg_kernel(pad_n, nch):
  mesh = plsc.VectorSubcoreMesh(core_axis_name="c", subcore_axis_name="s", num_cores=NC, num_subcores=NS)
  rows_per_tile = pad_n // NS

  @functools.partial(
      pl.kernel,
      out_type=[jax.ShapeDtypeStruct((pad_n,), jnp.float32)] * NC,
      mesh=mesh,
      scratch_types=[
          pltpu.VMEM((nch, CHUNK), jnp.int32),
          pltpu.VMEM((CHUNK,), jnp.float32),
          pltpu.VMEM((rows_per_tile,), jnp.float32),
          pltpu.VMEM_SHARED((pad_n,), jnp.float32),
      ],
  )
  def deg_kernel(dst_hbm, out0_hbm, out1_hbm, didx, ones_v, zbuf, acc):
    c = lax.axis_index("c")
    s = lax.axis_index("s")
    wid = s * NC + c
    z = jnp.zeros((16,), jnp.float32)
    for i in range(rows_per_tile // 16):
      zbuf[pl.ds(i * 16, 16)] = z
    one = jnp.ones((16,), jnp.float32)
    for i in range(CHUNK // 16):
      ones_v[pl.ds(i * 16, 16)] = one
    pltpu.sync_copy(zbuf, acc.at[pl.ds(s * rows_per_tile, rows_per_tile)])
    plsc.subcore_barrier()
    pltpu.sync_copy(dst_hbm.at[wid], didx)

    def body(j, _):
      pltpu.sync_copy(ones_v, acc.at[didx.at[j]], add=True)
      return ()

    lax.fori_loop(0, nch, body, ())
    plsc.subcore_barrier()
    sl = pl.ds(s * rows_per_tile, rows_per_tile)

    @pl.when(c == 0)
    def _():
      pltpu.sync_copy(acc.at[sl], out0_hbm.at[sl])

    @pl.when(c == 1)
    def _():
      pltpu.sync_copy(acc.at[sl], out1_hbm.at[sl])

  return deg_kernel


def _make_scatter_kernel(pad_n, nh, d, ncores, nseg):
  mesh = plsc.VectorSubcoreMesh(core_axis_name="c", subcore_axis_name="s", num_cores=ncores, num_subcores=NS)
  rows_per_tile = pad_n // NS

  @functools.partial(
      pl.kernel,
      out_type=jax.ShapeDtypeStruct((ncores, pad_n, d), jnp.float32),
      mesh=mesh,
      scratch_types=[
          pltpu.VMEM((nh, CHUNK), jnp.int32),
          pltpu.VMEM((nh, CHUNK), jnp.int32),
          pltpu.VMEM((CHUNK, d), jnp.float32),
          pltpu.VMEM((CHUNK, d), jnp.float32),
          pltpu.VMEM((16, d), jnp.float32),
          pltpu.VMEM_SHARED((pad_n, d), jnp.float32),
          pltpu.SemaphoreType.DMA,
          pltpu.SemaphoreType.DMA,
      ],
  )
  def scatter_kernel(g_hbm, src_hbm, dst_hbm, out_hbm,
                     sidx, didx, rows_a, rows_b, zbuf, acc, sem_a, sem_b):
    c = lax.axis_index("c")
    s = lax.axis_index("s")
    wid = s * ncores + c
    _zero_vmem(zbuf, 16, d)
    for k in range(rows_per_tile // 16):
      pltpu.sync_copy(zbuf, acc.at[pl.ds(s * rows_per_tile + k * 16, 16)])
    plsc.subcore_barrier()

    gather = lambda j, buf, sem: pltpu.async_copy(g_hbm.at[didx.at[j]], buf, sem)

    for h in range(nseg):
      pltpu.sync_copy(src_hbm.at[wid].at[pl.ds(h * nh, nh)], sidx)
      pltpu.sync_copy(dst_hbm.at[wid].at[pl.ds(h * nh, nh)], didx)

      def body(j, _):
        gather(j, rows_a, sem_a).wait()
        pltpu.sync_copy(rows_a, acc.at[sidx.at[j]], add=True)
        return ()

      lax.fori_loop(0, nh, body, ())
      del rows_b, sem_b

    plsc.subcore_barrier()
    pltpu.sync_copy(acc.at[pl.ds(s * rows_per_tile, rows_per_tile)],
                    out_hbm.at[c].at[pl.ds(s * rows_per_tile, rows_per_tile)])

  return scatter_kernel


SCAT_CORES = 2
SCAT_NSEG = 1


def _dinv_from_deg(degp):
  deg = SELF_C + degp[0] + degp[1]
  return lax.rsqrt(deg)


def _sum_parts(part):
  agg = part[0]
  for i in range(1, part.shape[0]):
    agg = agg + part[i]
  return agg


def _tc_k1(x_ref, w1_ref, degp_ref, h1_ref, g1_ref):
  h = jnp.dot(x_ref[...], w1_ref[...], preferred_element_type=jnp.float32)
  dinv = _dinv_from_deg(degp_ref[...])
  h1_ref[...] = h
  g1_ref[...] = h * dinv


def _tc_k2(part_ref, h1_ref, degp_ref, b1_ref, w2_ref, h2_ref, g2_ref):
  dinv = _dinv_from_deg(degp_ref[...])
  h1 = h1_ref[...]
  agg = _sum_parts(part_ref[...])
  z = jnp.maximum(dinv * agg + SELF_C * dinv * dinv * h1 + b1_ref[...], 0.0)
  h2 = jnp.dot(z, w2_ref[...], preferred_element_type=jnp.float32)
  h2_ref[...] = h2
  g2_ref[...] = h2 * dinv


def _tc_k3(part_ref, h2_ref, degp_ref, b2_ref, out_ref):
  dinv = _dinv_from_deg(degp_ref[...])
  h2 = h2_ref[...]
  agg = _sum_parts(part_ref[...])
  out_ref[...] = dinv * agg + SELF_C * dinv * dinv * h2 + b2_ref[...]


def kernel(x, edge_index, W1, b1, W2, b2):
  n, d = x.shape
  e = edge_index.shape[1]

  pad_n = ((n + 1 + NS * 64 - 1) // (NS * 64)) * (NS * 64)
  src = edge_index[0].astype(jnp.int32)
  dst = edge_index[1].astype(jnp.int32)

  nch_d = (e + NW * CHUNK - 1) // (NW * CHUNK)
  ep_d = NW * CHUNK * nch_d
  dst_d = jnp.pad(dst, (0, ep_d - e), constant_values=n).reshape(NW, nch_d, CHUNK)

  sw = SCAT_CORES * NS
  raw = (e + sw * CHUNK - 1) // (sw * CHUNK)
  nh = ((raw + SCAT_NSEG - 1) // SCAT_NSEG + 7) // 8 * 8
  nch_s = SCAT_NSEG * nh
  ep_s = sw * CHUNK * nch_s
  src_s = jnp.pad(src, (0, ep_s - e), constant_values=n).reshape(sw, nch_s, CHUNK)
  dst_s = jnp.pad(dst, (0, ep_s - e), constant_values=n).reshape(sw, nch_s, CHUNK)

  x_pad = jnp.pad(x, ((0, pad_n - n), (0, 0)))
  b1r = b1.reshape(1, d)
  b2r = b2.reshape(1, d)

  deg_kernel = _make_deg_kernel(pad_n, nch_d)
  scatter_kernel = _make_scatter_kernel(pad_n, nh, d, SCAT_CORES, SCAT_NSEG)

  deg0, deg1 = deg_kernel(dst_d)
  degp = jnp.stack([deg0, deg1]).reshape(NC, pad_n, 1)

  bm = 512
  grid = (pad_n // bm,)
  spec_x = pl.BlockSpec((bm, d), lambda m: (m, 0))
  spec_w = pl.BlockSpec((d, d), lambda m: (0, 0))
  spec_deg = pl.BlockSpec((NC, bm, 1), lambda m: (0, m, 0))
  spec_part = pl.BlockSpec((SCAT_CORES, bm, d), lambda m: (0, m, 0))
  spec_b = pl.BlockSpec((1, d), lambda m: (0, 0))

  h1, g1 = pl.pallas_call(
      _tc_k1,
      grid=grid,
      in_specs=[spec_x, spec_w, spec_deg],
      out_specs=[spec_x, spec_x],
      out_shape=[jax.ShapeDtypeStruct((pad_n, d), jnp.float32)] * 2,
  )(x_pad, W1, degp)

  part1 = scatter_kernel(g1, src_s, dst_s)

  h2, g2 = pl.pallas_call(
      _tc_k2,
      grid=grid,
      in_specs=[spec_part, spec_x, spec_deg, spec_b, spec_w],
      out_specs=[spec_x, spec_x],
      out_shape=[jax.ShapeDtypeStruct((pad_n, d), jnp.float32)] * 2,
  )(part1, h1, degp, b1r, W2)

  part2 = scatter_kernel(g2, src_s, dst_s)

  out = pl.pallas_call(
      _tc_k3,
      grid=grid,
      in_specs=[spec_part, spec_x, spec_deg, spec_b],
      out_specs=spec_x,
      out_shape=jax.ShapeDtypeStruct((pad_n, d), jnp.float32),
  )(part2, h2, degp, b2r)

  return out[:n]


# speedup vs baseline: 1.4460x; 1.4460x over previous
import functools

import jax
import jax.numpy as jnp
from jax import lax
from jax.experimental import pallas as pl
from jax.experimental.pallas import tpu as pltpu
import jax.experimental.pallas.tpu_sc as plsc

SELF_C = 1.0

NC = 2
NS = 16
NW = NC * NS

CHUNK = 128


def _zero_vmem(buf, rows, width, dtype=jnp.float32):
  z = jnp.zeros((16,), dtype)
  for i in range(rows):
    for j in range(width // 16):
      buf[i, pl.ds(j * 16, 16)] = z


def _make_deg_kernel(pad_n, nch):
  mesh = plsc.VectorSubcoreMesh(core_axis_name="c", subcore_axis_name="s", num_cores=NC, num_subcores=NS)
  rows_per_tile = pad_n // NS

  @functools.partial(
      pl.kernel,
      out_type=[jax.ShapeDtypeStruct((pad_n,), jnp.float32)] * NC,
      mesh=mesh,
      scratch_types=[
          pltpu.VMEM((nch, CHUNK), jnp.int32),
          pltpu.VMEM((CHUNK,), jnp.float32),
          pltpu.VMEM((rows_per_tile,), jnp.float32),
          pltpu.VMEM_SHARED((pad_n,), jnp.float32),
      ],
  )
  def deg_kernel(dst_hbm, out0_hbm, out1_hbm, didx, ones_v, zbuf, acc):
    c = lax.axis_index("c")
    s = lax.axis_index("s")
    wid = s * NC + c
    z = jnp.zeros((16,), jnp.float32)
    for i in range(rows_per_tile // 16):
      zbuf[pl.ds(i * 16, 16)] = z
    one = jnp.ones((16,), jnp.float32)
    for i in range(CHUNK // 16):
      ones_v[pl.ds(i * 16, 16)] = one
    pltpu.sync_copy(zbuf, acc.at[pl.ds(s * rows_per_tile, rows_per_tile)])
    plsc.subcore_barrier()
    pltpu.sync_copy(dst_hbm.at[wid], didx)

    def body(j, _):
      pltpu.sync_copy(ones_v, acc.at[didx.at[j]], add=True)
      return ()

    lax.fori_loop(0, nch, body, ())
    plsc.subcore_barrier()
    sl = pl.ds(s * rows_per_tile, rows_per_tile)

    @pl.when(c == 0)
    def _():
      pltpu.sync_copy(acc.at[sl], out0_hbm.at[sl])

    @pl.when(c == 1)
    def _():
      pltpu.sync_copy(acc.at[sl], out1_hbm.at[sl])

  return deg_kernel


def _make_scatter_kernel(pad_n, nh, d, ncores, nseg):
  mesh = plsc.VectorSubcoreMesh(core_axis_name="c", subcore_axis_name="s", num_cores=ncores, num_subcores=NS)
  rows_per_tile = pad_n // NS

  @functools.partial(
      pl.kernel,
      out_type=jax.ShapeDtypeStruct((ncores, pad_n, d), jnp.float32),
      mesh=mesh,
      scratch_types=[
          pltpu.VMEM((nseg * nh, CHUNK), jnp.int32),
          pltpu.VMEM((nseg * nh, CHUNK), jnp.int32),
          pltpu.VMEM((CHUNK, d), jnp.float32),
          pltpu.VMEM((64, d), jnp.float32),
          pltpu.VMEM_SHARED((pad_n, d), jnp.float32),
          pltpu.SemaphoreType.DMA,
      ],
  )
  def scatter_kernel(g_hbm, src_hbm, dst_hbm, out_hbm,
                     sidx, didx, rows, zbuf, acc, sem):
    c = lax.axis_index("c")
    s = lax.axis_index("s")
    wid = s * ncores + c
    _zero_vmem(zbuf, 64, d)
    for k in range(rows_per_tile // 64):
      pltpu.sync_copy(zbuf, acc.at[pl.ds(s * rows_per_tile + k * 64, 64)])
    plsc.subcore_barrier()
    pltpu.sync_copy(src_hbm.at[wid], sidx)
    pltpu.sync_copy(dst_hbm.at[wid], didx)

    def body(j, _):
      pltpu.async_copy(g_hbm.at[didx.at[j]], rows, sem).wait()
      pltpu.sync_copy(rows, acc.at[sidx.at[j]], add=True)
      return ()

    lax.fori_loop(0, nseg * nh, body, ())

    plsc.subcore_barrier()
    pltpu.sync_copy(acc.at[pl.ds(s * rows_per_tile, rows_per_tile)],
                    out_hbm.at[c].at[pl.ds(s * rows_per_tile, rows_per_tile)])

  return scatter_kernel


SCAT_CORES = 2
SCAT_NSEG = 1


def _dinv_from_deg(degp):
  deg = SELF_C + degp[0] + degp[1]
  return lax.rsqrt(deg)


def _sum_parts(part):
  agg = part[0]
  for i in range(1, part.shape[0]):
    agg = agg + part[i]
  return agg


def _tc_k1(x_ref, w1_ref, degp_ref, h1_ref, g1_ref):
  h = jnp.dot(x_ref[...], w1_ref[...], preferred_element_type=jnp.float32)
  dinv = _dinv_from_deg(degp_ref[...])
  h1_ref[...] = h
  g1_ref[...] = h * dinv


def _tc_k2(part_ref, h1_ref, degp_ref, b1_ref, w2_ref, h2_ref, g2_ref):
  dinv = _dinv_from_deg(degp_ref[...])
  h1 = h1_ref[...]
  agg = _sum_parts(part_ref[...])
  z = jnp.maximum(dinv * agg + SELF_C * dinv * dinv * h1 + b1_ref[...], 0.0)
  h2 = jnp.dot(z, w2_ref[...], preferred_element_type=jnp.float32)
  h2_ref[...] = h2
  g2_ref[...] = h2 * dinv


def _tc_k3(part_ref, h2_ref, degp_ref, b2_ref, out_ref):
  dinv = _dinv_from_deg(degp_ref[...])
  h2 = h2_ref[...]
  agg = _sum_parts(part_ref[...])
  out_ref[...] = dinv * agg + SELF_C * dinv * dinv * h2 + b2_ref[...]


def kernel(x, edge_index, W1, b1, W2, b2):
  n, d = x.shape
  e = edge_index.shape[1]

  pad_n = ((n + 1 + NS * 64 - 1) // (NS * 64)) * (NS * 64)
  src = edge_index[0].astype(jnp.int32)
  dst = edge_index[1].astype(jnp.int32)

  nch_d = (e + NW * CHUNK - 1) // (NW * CHUNK)
  ep_d = NW * CHUNK * nch_d
  dst_d = jnp.pad(dst, (0, ep_d - e), constant_values=n).reshape(NW, nch_d, CHUNK)

  sw = SCAT_CORES * NS
  raw = (e + sw * CHUNK - 1) // (sw * CHUNK)
  nh = (raw + SCAT_NSEG - 1) // SCAT_NSEG
  if SCAT_NSEG > 1:
    nh = (nh + 7) // 8 * 8
  nch_s = SCAT_NSEG * nh
  ep_s = sw * CHUNK * nch_s
  src_s = jnp.pad(src, (0, ep_s - e), constant_values=n).reshape(sw, nch_s, CHUNK)
  dst_s = jnp.pad(dst, (0, ep_s - e), constant_values=n).reshape(sw, nch_s, CHUNK)

  x_pad = jnp.pad(x, ((0, pad_n - n), (0, 0)))
  b1r = b1.reshape(1, d)
  b2r = b2.reshape(1, d)

  deg_kernel = _make_deg_kernel(pad_n, nch_d)
  scatter_kernel = _make_scatter_kernel(pad_n, nh, d, SCAT_CORES, SCAT_NSEG)

  deg0, deg1 = deg_kernel(dst_d)
  degp = jnp.stack([deg0, deg1]).reshape(NC, pad_n, 1)

  bm = 512
  grid = (pad_n // bm,)
  spec_x = pl.BlockSpec((bm, d), lambda m: (m, 0))
  spec_w = pl.BlockSpec((d, d), lambda m: (0, 0))
  spec_deg = pl.BlockSpec((NC, bm, 1), lambda m: (0, m, 0))
  spec_part = pl.BlockSpec((SCAT_CORES, bm, d), lambda m: (0, m, 0))
  spec_b = pl.BlockSpec((1, d), lambda m: (0, 0))

  h1, g1 = pl.pallas_call(
      _tc_k1,
      grid=grid,
      in_specs=[spec_x, spec_w, spec_deg],
      out_specs=[spec_x, spec_x],
      out_shape=[jax.ShapeDtypeStruct((pad_n, d), jnp.float32)] * 2,
  )(x_pad, W1, degp)

  part1 = scatter_kernel(g1, src_s, dst_s)

  h2, g2 = pl.pallas_call(
      _tc_k2,
      grid=grid,
      in_specs=[spec_part, spec_x, spec_deg, spec_b, spec_w],
      out_specs=[spec_x, spec_x],
      out_shape=[jax.ShapeDtypeStruct((pad_n, d), jnp.float32)] * 2,
  )(part1, h1, degp, b1r, W2)

  part2 = scatter_kernel(g2, src_s, dst_s)

  out = pl.pallas_call(
      _tc_k3,
      grid=grid,
      in_specs=[spec_part, spec_x, spec_deg, spec_b],
      out_specs=spec_x,
      out_shape=jax.ShapeDtypeStruct((pad_n, d), jnp.float32),
  )(part2, h2, degp, b2r)

  return out[:n]


# speedup vs baseline: 2.6622x; 1.8411x over previous
import functools

import jax
import jax.numpy as jnp
from jax import lax
from jax.experimental import pallas as pl
from jax.experimental.pallas import tpu as pltpu
import jax.experimental.pallas.tpu_sc as plsc

SELF_C = 1.0

NC = 2
NS = 16
NW = NC * NS

CHUNK = 128


def _zero_vmem(buf, rows, width, dtype=jnp.float32):
  z = jnp.zeros((16,), dtype)
  for i in range(rows):
    for j in range(width // 16):
      buf[i, pl.ds(j * 16, 16)] = z


def _make_deg_kernel(pad_n, nch):
  mesh = plsc.VectorSubcoreMesh(core_axis_name="c", subcore_axis_name="s", num_cores=NC, num_subcores=NS)
  rows_per_tile = pad_n // NS

  @functools.partial(
      pl.kernel,
      out_type=[jax.ShapeDtypeStruct((pad_n,), jnp.float32)] * NC,
      mesh=mesh,
      scratch_types=[
          pltpu.VMEM((nch, CHUNK), jnp.int32),
          pltpu.VMEM((CHUNK,), jnp.float32),
          pltpu.VMEM((rows_per_tile,), jnp.float32),
          pltpu.VMEM_SHARED((pad_n,), jnp.float32),
      ],
  )
  def deg_kernel(dst_hbm, out0_hbm, out1_hbm, didx, ones_v, zbuf, acc):
    c = lax.axis_index("c")
    s = lax.axis_index("s")
    wid = s * NC + c
    z = jnp.zeros((16,), jnp.float32)
    for i in range(rows_per_tile // 16):
      zbuf[pl.ds(i * 16, 16)] = z
    one = jnp.ones((16,), jnp.float32)
    for i in range(CHUNK // 16):
      ones_v[pl.ds(i * 16, 16)] = one
    pltpu.sync_copy(zbuf, acc.at[pl.ds(s * rows_per_tile, rows_per_tile)])
    plsc.subcore_barrier()
    pltpu.sync_copy(dst_hbm.at[wid], didx)

    def body(j, _):
      pltpu.sync_copy(ones_v, acc.at[didx.at[j]], add=True)
      return ()

    lax.fori_loop(0, nch, body, ())
    plsc.subcore_barrier()
    sl = pl.ds(s * rows_per_tile, rows_per_tile)

    @pl.when(c == 0)
    def _():
      pltpu.sync_copy(acc.at[sl], out0_hbm.at[sl])

    @pl.when(c == 1)
    def _():
      pltpu.sync_copy(acc.at[sl], out1_hbm.at[sl])

  return deg_kernel


def _make_scatter_kernel(pad_n, nh, d, ncores, nseg):
  mesh = plsc.VectorSubcoreMesh(core_axis_name="c", subcore_axis_name="s", num_cores=ncores, num_subcores=NS)
  rows_per_tile = pad_n // NS

  @functools.partial(
      pl.kernel,
      out_type=jax.ShapeDtypeStruct((ncores, pad_n, d), jnp.float32),
      mesh=mesh,
      scratch_types=[
          pltpu.VMEM((nseg * nh, CHUNK), jnp.int32),
          pltpu.VMEM((nseg * nh, CHUNK), jnp.int32),
          pltpu.VMEM((CHUNK, d), jnp.float32),
          pltpu.VMEM((64, d), jnp.float32),
          pltpu.VMEM_SHARED((pad_n, d), jnp.float32),
          pltpu.SemaphoreType.DMA,
      ],
  )
  def scatter_kernel(g_hbm, src_hbm, dst_hbm, out_hbm,
                     sidx, didx, rows, zbuf, acc, sem):
    c = lax.axis_index("c")
    s = lax.axis_index("s")
    wid = s * ncores + c
    _zero_vmem(zbuf, 64, d)
    for k in range(rows_per_tile // 64):
      pltpu.sync_copy(zbuf, acc.at[pl.ds(s * rows_per_tile + k * 64, 64)])
    plsc.subcore_barrier()
    pltpu.sync_copy(src_hbm.at[wid], sidx)
    pltpu.sync_copy(dst_hbm.at[wid], didx)

    def body(j, _):
      pltpu.async_copy(g_hbm.at[didx.at[j]], rows, sem).wait()
      pltpu.sync_copy(rows, acc.at[sidx.at[j]], add=True)
      return ()

    lax.fori_loop(0, nseg * nh, body, ())

    plsc.subcore_barrier()
    pltpu.sync_copy(acc.at[pl.ds(s * rows_per_tile, rows_per_tile)],
                    out_hbm.at[c].at[pl.ds(s * rows_per_tile, rows_per_tile)])

  return scatter_kernel


SCAT_CORES = 2
SCAT_NSEG = 1


def _dinv_from_deg(degp):
  deg = SELF_C + degp[0] + degp[1]
  return lax.rsqrt(deg)


def _sum_parts(part):
  agg = part[0]
  for i in range(1, part.shape[0]):
    agg = agg + part[i]
  return agg


def _row_mask(n, bm):
  m = pl.program_id(0)
  rows = m * bm + lax.broadcasted_iota(jnp.int32, (bm, 1), 0)
  return rows < n


def _tc_k1(x_ref, w1_ref, degp_ref, h1_ref, g1_ref, *, n, bm):
  h = jnp.dot(x_ref[...], w1_ref[...], preferred_element_type=jnp.float32)
  dinv = _dinv_from_deg(degp_ref[...])
  h1_ref[...] = h
  g1_ref[...] = jnp.where(_row_mask(n, bm), h * dinv, 0.0)


def _tc_k2(part_ref, h1_ref, degp_ref, b1_ref, w2_ref, h2_ref, g2_ref, *, n, bm):
  dinv = _dinv_from_deg(degp_ref[...])
  h1 = h1_ref[...]
  agg = _sum_parts(part_ref[...])
  z = jnp.maximum(dinv * agg + SELF_C * dinv * dinv * h1 + b1_ref[...], 0.0)
  h2 = jnp.dot(z, w2_ref[...], preferred_element_type=jnp.float32)
  h2_ref[...] = h2
  g2_ref[...] = jnp.where(_row_mask(n, bm), h2 * dinv, 0.0)


def _tc_k3(part_ref, h2_ref, degp_ref, b2_ref, out_ref):
  dinv = _dinv_from_deg(degp_ref[...])
  h2 = h2_ref[...]
  agg = _sum_parts(part_ref[...])
  out_ref[...] = dinv * agg + SELF_C * dinv * dinv * h2 + b2_ref[...]


def kernel(x, edge_index, W1, b1, W2, b2):
  n, d = x.shape
  e = edge_index.shape[1]

  pad_n = ((n + 1 + NS * 64 - 1) // (NS * 64)) * (NS * 64)
  src = edge_index[0].astype(jnp.int32)
  dst = edge_index[1].astype(jnp.int32)

  nch_d = (e + NW * CHUNK - 1) // (NW * CHUNK)
  ep_d = NW * CHUNK * nch_d
  dst_d = jnp.pad(dst, (0, ep_d - e), constant_values=n).reshape(NW, nch_d, CHUNK)

  sw = SCAT_CORES * NS
  raw = (e + sw * CHUNK - 1) // (sw * CHUNK)
  nh = (raw + SCAT_NSEG - 1) // SCAT_NSEG
  if SCAT_NSEG > 1:
    nh = (nh + 7) // 8 * 8
  nch_s = SCAT_NSEG * nh
  ep_s = sw * CHUNK * nch_s
  spare = jnp.arange(ep_s - e, dtype=jnp.int32) % (pad_n - n) + n
  src_s = jnp.concatenate([src, spare]).reshape(sw, nch_s, CHUNK)
  dst_s = jnp.concatenate([dst, spare]).reshape(sw, nch_s, CHUNK)

  x_pad = jnp.pad(x, ((0, pad_n - n), (0, 0)))
  b1r = b1.reshape(1, d)
  b2r = b2.reshape(1, d)

  deg_kernel = _make_deg_kernel(pad_n, nch_d)
  scatter_kernel = _make_scatter_kernel(pad_n, nh, d, SCAT_CORES, SCAT_NSEG)

  deg0, deg1 = deg_kernel(dst_d)
  degp = jnp.stack([deg0, deg1]).reshape(NC, pad_n, 1)

  bm = 512
  grid = (pad_n // bm,)
  spec_x = pl.BlockSpec((bm, d), lambda m: (m, 0))
  spec_w = pl.BlockSpec((d, d), lambda m: (0, 0))
  spec_deg = pl.BlockSpec((NC, bm, 1), lambda m: (0, m, 0))
  spec_part = pl.BlockSpec((SCAT_CORES, bm, d), lambda m: (0, m, 0))
  spec_b = pl.BlockSpec((1, d), lambda m: (0, 0))

  h1, g1 = pl.pallas_call(
      functools.partial(_tc_k1, n=n, bm=bm),
      grid=grid,
      in_specs=[spec_x, spec_w, spec_deg],
      out_specs=[spec_x, spec_x],
      out_shape=[jax.ShapeDtypeStruct((pad_n, d), jnp.float32)] * 2,
  )(x_pad, W1, degp)

  part1 = scatter_kernel(g1, src_s, dst_s)

  h2, g2 = pl.pallas_call(
      functools.partial(_tc_k2, n=n, bm=bm),
      grid=grid,
      in_specs=[spec_part, spec_x, spec_deg, spec_b, spec_w],
      out_specs=[spec_x, spec_x],
      out_shape=[jax.ShapeDtypeStruct((pad_n, d), jnp.float32)] * 2,
  )(part1, h1, degp, b1r, W2)

  part2 = scatter_kernel(g2, src_s, dst_s)

  out = pl.pallas_call(
      _tc_k3,
      grid=grid,
      in_specs=[spec_part, spec_x, spec_deg, spec_b],
      out_specs=spec_x,
      out_shape=jax.ShapeDtypeStruct((pad_n, d), jnp.float32),
  )(part2, h2, degp, b2r)

  return out[:n]


# speedup vs baseline: 3.6907x; 1.3864x over previous
import functools

import jax
import jax.numpy as jnp
from jax import lax
from jax.experimental import pallas as pl
from jax.experimental.pallas import tpu as pltpu
import jax.experimental.pallas.tpu_sc as plsc

SELF_C = 1.0

NC = 2
NS = 16
NW = NC * NS

CHUNK = 128


def _zero_vmem(buf, rows, width, dtype=jnp.float32):
  z = jnp.zeros((16,), dtype)
  for i in range(rows):
    for j in range(width // 16):
      buf[i, pl.ds(j * 16, 16)] = z


def _make_deg_kernel(pad_n, nch):
  mesh = plsc.VectorSubcoreMesh(core_axis_name="c", subcore_axis_name="s", num_cores=NC, num_subcores=NS)
  rows_per_tile = pad_n // NS

  @functools.partial(
      pl.kernel,
      out_type=[jax.ShapeDtypeStruct((pad_n,), jnp.float32)] * NC,
      mesh=mesh,
      scratch_types=[
          pltpu.VMEM((nch, CHUNK), jnp.int32),
          pltpu.VMEM((CHUNK,), jnp.float32),
          pltpu.VMEM((rows_per_tile,), jnp.float32),
          pltpu.VMEM_SHARED((pad_n,), jnp.float32),
      ],
  )
  def deg_kernel(dst_hbm, out0_hbm, out1_hbm, didx, ones_v, zbuf, acc):
    c = lax.axis_index("c")
    s = lax.axis_index("s")
    wid = s * NC + c
    z = jnp.zeros((16,), jnp.float32)
    for i in range(rows_per_tile // 16):
      zbuf[pl.ds(i * 16, 16)] = z
    one = jnp.ones((16,), jnp.float32)
    for i in range(CHUNK // 16):
      ones_v[pl.ds(i * 16, 16)] = one
    pltpu.sync_copy(zbuf, acc.at[pl.ds(s * rows_per_tile, rows_per_tile)])
    plsc.subcore_barrier()
    pltpu.sync_copy(dst_hbm.at[wid], didx)

    def body(j, _):
      pltpu.sync_copy(ones_v, acc.at[didx.at[j]], add=True)
      return ()

    lax.fori_loop(0, nch, body, ())
    plsc.subcore_barrier()
    sl = pl.ds(s * rows_per_tile, rows_per_tile)

    @pl.when(c == 0)
    def _():
      pltpu.sync_copy(acc.at[sl], out0_hbm.at[sl])

    @pl.when(c == 1)
    def _():
      pltpu.sync_copy(acc.at[sl], out1_hbm.at[sl])

  return deg_kernel


def _make_scatter_kernel(pad_n, nh, d, ncores, nseg):
  mesh = plsc.VectorSubcoreMesh(core_axis_name="c", subcore_axis_name="s", num_cores=ncores, num_subcores=NS)
  rows_per_tile = pad_n // NS

  @functools.partial(
      pl.kernel,
      out_type=jax.ShapeDtypeStruct((ncores, pad_n, d), jnp.float32),
      mesh=mesh,
      scratch_types=[
          pltpu.VMEM((nh, CHUNK), jnp.int32),
          pltpu.VMEM((nh, CHUNK), jnp.int32),
          pltpu.VMEM((CHUNK, d), jnp.float32),
          pltpu.VMEM((CHUNK, d), jnp.float32),
          pltpu.VMEM((16, d), jnp.float32),
          pltpu.VMEM_SHARED((pad_n, d), jnp.float32),
          pltpu.SemaphoreType.DMA,
          pltpu.SemaphoreType.DMA,
      ],
  )
  def scatter_kernel(g_hbm, src_hbm, dst_hbm, out_hbm,
                     sidx, didx, rows_a, rows_b, zbuf, acc, sem_a, sem_b):
    c = lax.axis_index("c")
    s = lax.axis_index("s")
    wid = s * ncores + c
    _zero_vmem(zbuf, 16, d)
    for k in range(rows_per_tile // 16):
      pltpu.sync_copy(zbuf, acc.at[pl.ds(s * rows_per_tile + k * 16, 16)])
    plsc.subcore_barrier()

    gather = lambda j, buf, sem: pltpu.async_copy(g_hbm.at[didx.at[j]], buf, sem)

    for h in range(nseg):
      pltpu.sync_copy(src_hbm.at[wid].at[pl.ds(h * nh, nh)], sidx)
      pltpu.sync_copy(dst_hbm.at[wid].at[pl.ds(h * nh, nh)], didx)

      gather(0, rows_a, sem_a)

      def body(k, _):
        j0 = 2 * k
        gather(j0 + 1, rows_b, sem_b)
        pltpu.make_async_copy(g_hbm.at[didx.at[j0]], rows_a, sem_a).wait()
        pltpu.sync_copy(rows_a, acc.at[sidx.at[j0]], add=True)

        @pl.when(j0 + 2 < nh)
        def _():
          gather(j0 + 2, rows_a, sem_a)

        pltpu.make_async_copy(g_hbm.at[didx.at[j0 + 1]], rows_b, sem_b).wait()
        pltpu.sync_copy(rows_b, acc.at[sidx.at[j0 + 1]], add=True)
        return ()

      lax.fori_loop(0, nh // 2, body, ())

    plsc.subcore_barrier()
    pltpu.sync_copy(acc.at[pl.ds(s * rows_per_tile, rows_per_tile)],
                    out_hbm.at[c].at[pl.ds(s * rows_per_tile, rows_per_tile)])

  return scatter_kernel


SCAT_CORES = 2
SCAT_NSEG = 2


def _dinv_from_deg(degp):
  deg = SELF_C + degp[0] + degp[1]
  return lax.rsqrt(deg)


def _sum_parts(part):
  agg = part[0]
  for i in range(1, part.shape[0]):
    agg = agg + part[i]
  return agg


def _row_mask(n, bm):
  m = pl.program_id(0)
  rows = m * bm + lax.broadcasted_iota(jnp.int32, (bm, 1), 0)
  return rows < n


def _tc_k1(x_ref, w1_ref, degp_ref, h1_ref, g1_ref, *, n, bm):
  h = jnp.dot(x_ref[...], w1_ref[...], preferred_element_type=jnp.float32)
  dinv = _dinv_from_deg(degp_ref[...])
  h1_ref[...] = h
  g1_ref[...] = jnp.where(_row_mask(n, bm), h * dinv, 0.0)


def _tc_k2(part_ref, h1_ref, degp_ref, b1_ref, w2_ref, h2_ref, g2_ref, *, n, bm):
  dinv = _dinv_from_deg(degp_ref[...])
  h1 = h1_ref[...]
  agg = _sum_parts(part_ref[...])
  z = jnp.maximum(dinv * agg + SELF_C * dinv * dinv * h1 + b1_ref[...], 0.0)
  h2 = jnp.dot(z, w2_ref[...], preferred_element_type=jnp.float32)
  h2_ref[...] = h2
  g2_ref[...] = jnp.where(_row_mask(n, bm), h2 * dinv, 0.0)


def _tc_k3(part_ref, h2_ref, degp_ref, b2_ref, out_ref):
  dinv = _dinv_from_deg(degp_ref[...])
  h2 = h2_ref[...]
  agg = _sum_parts(part_ref[...])
  out_ref[...] = dinv * agg + SELF_C * dinv * dinv * h2 + b2_ref[...]


def kernel(x, edge_index, W1, b1, W2, b2):
  n, d = x.shape
  e = edge_index.shape[1]

  pad_n = ((n + 1 + NS * 64 - 1) // (NS * 64)) * (NS * 64)
  src = edge_index[0].astype(jnp.int32)
  dst = edge_index[1].astype(jnp.int32)

  nch_d = (e + NW * CHUNK - 1) // (NW * CHUNK)
  ep_d = NW * CHUNK * nch_d
  dst_d = jnp.pad(dst, (0, ep_d - e), constant_values=n).reshape(NW, nch_d, CHUNK)

  sw = SCAT_CORES * NS
  raw = (e + sw * CHUNK - 1) // (sw * CHUNK)
  nh = (raw + SCAT_NSEG - 1) // SCAT_NSEG
  if SCAT_NSEG > 1:
    nh = (nh + 7) // 8 * 8
  nch_s = SCAT_NSEG * nh
  ep_s = sw * CHUNK * nch_s
  spare = jnp.arange(ep_s - e, dtype=jnp.int32) % (pad_n - n) + n
  src_s = jnp.concatenate([src, spare]).reshape(sw, nch_s, CHUNK)
  dst_s = jnp.concatenate([dst, spare]).reshape(sw, nch_s, CHUNK)

  x_pad = jnp.pad(x, ((0, pad_n - n), (0, 0)))
  b1r = b1.reshape(1, d)
  b2r = b2.reshape(1, d)

  deg_kernel = _make_deg_kernel(pad_n, nch_d)
  scatter_kernel = _make_scatter_kernel(pad_n, nh, d, SCAT_CORES, SCAT_NSEG)

  deg0, deg1 = deg_kernel(dst_d)
  degp = jnp.stack([deg0, deg1]).reshape(NC, pad_n, 1)

  bm = 512
  grid = (pad_n // bm,)
  spec_x = pl.BlockSpec((bm, d), lambda m: (m, 0))
  spec_w = pl.BlockSpec((d, d), lambda m: (0, 0))
  spec_deg = pl.BlockSpec((NC, bm, 1), lambda m: (0, m, 0))
  spec_part = pl.BlockSpec((SCAT_CORES, bm, d), lambda m: (0, m, 0))
  spec_b = pl.BlockSpec((1, d), lambda m: (0, 0))

  h1, g1 = pl.pallas_call(
      functools.partial(_tc_k1, n=n, bm=bm),
      grid=grid,
      in_specs=[spec_x, spec_w, spec_deg],
      out_specs=[spec_x, spec_x],
      out_shape=[jax.ShapeDtypeStruct((pad_n, d), jnp.float32)] * 2,
  )(x_pad, W1, degp)

  part1 = scatter_kernel(g1, src_s, dst_s)

  h2, g2 = pl.pallas_call(
      functools.partial(_tc_k2, n=n, bm=bm),
      grid=grid,
      in_specs=[spec_part, spec_x, spec_deg, spec_b, spec_w],
      out_specs=[spec_x, spec_x],
      out_shape=[jax.ShapeDtypeStruct((pad_n, d), jnp.float32)] * 2,
  )(part1, h1, degp, b1r, W2)

  part2 = scatter_kernel(g2, src_s, dst_s)

  out = pl.pallas_call(
      _tc_k3,
      grid=grid,
      in_specs=[spec_part, spec_x, spec_deg, spec_b],
      out_specs=spec_x,
      out_shape=jax.ShapeDtypeStruct((pad_n, d), jnp.float32),
  )(part2, h2, degp, b2r)

  return out[:n]
